# initial kernel scaffold (unmeasured)
import jax
import jax.numpy as jnp
from jax import lax
from jax.experimental import pallas as pl
from jax.experimental.pallas import tpu as pltpu

M = 2048
N = 2048
F_CHUNK = 2048


def kernel(dy, W):
    my_x = lax.axis_index("x")
    my_z = lax.axis_index("z")
    r = my_x * 2 + my_z
    dy_c = lax.dynamic_slice_in_dim(dy, r * F_CHUNK, F_CHUNK, axis=1)
    w_c = lax.dynamic_slice_in_dim(W, r * F_CHUNK, F_CHUNK, axis=1)

    def body(dy_ref, w_ref, out_ref, recv_ref, send_sems, recv_sems):
        x = lax.axis_index("x")
        y = lax.axis_index("y")
        z = lax.axis_index("z")

        barrier_sem = pltpu.get_barrier_semaphore()
        peers = [
            (1 - x, y, z),
            (x, 1 - y, z),
            (x, y, 1 - z),
        ]
        for peer in peers:
            pl.semaphore_signal(
                barrier_sem, inc=1,
                device_id=peer, device_id_type=pl.DeviceIdType.MESH,
            )
        pl.semaphore_wait(barrier_sem, 3)

        out_ref[...] = lax.dot_general(
            dy_ref[...], w_ref[...],
            dimension_numbers=(((1,), (1,)), ((), ())),
            preferred_element_type=jnp.float32,
        )

        for ph, peer in enumerate(peers):
            rdma = pltpu.make_async_remote_copy(
                src_ref=out_ref,
                dst_ref=recv_ref.at[ph],
                send_sem=send_sems.at[ph],
                recv_sem=recv_sems.at[ph],
                device_id=peer,
                device_id_type=pl.DeviceIdType.MESH,
            )
            rdma.start()
            rdma.wait()
            out_ref[...] += recv_ref[ph]

    return pl.pallas_call(
        body,
        out_shape=jax.ShapeDtypeStruct((M, N), jnp.float32),
        in_specs=[
            pl.BlockSpec(memory_space=pltpu.VMEM),
            pl.BlockSpec(memory_space=pltpu.VMEM),
        ],
        out_specs=pl.BlockSpec(memory_space=pltpu.VMEM),
        scratch_shapes=[
            pltpu.VMEM((3, M, N), jnp.float32),
            pltpu.SemaphoreType.DMA((3,)),
            pltpu.SemaphoreType.DMA((3,)),
        ],
        compiler_params=pltpu.CompilerParams(collective_id=0),
    )(dy_c, w_c)


# baseline (device time: 408341 ns/iter reference)
import jax
import jax.numpy as jnp
from jax import lax
from jax.experimental import pallas as pl
from jax.experimental.pallas import tpu as pltpu

M = 2048
N = 2048
F_CHUNK = 2048
BM = 512

H_X = M // 2
H_Y = M // 4
H_Z = M // 8


def _matmul(dy_c, w_c):
    def mm_body(dy_ref, w_ref, p_ref):
        p_ref[...] = lax.dot_general(
            dy_ref[...], w_ref[...],
            dimension_numbers=(((1,), (1,)), ((), ())),
            preferred_element_type=jnp.float32,
        )

    return pl.pallas_call(
        mm_body,
        grid=(M // BM, N // BM),
        in_specs=[
            pl.BlockSpec((BM, F_CHUNK), lambda i, j: (i, 0)),
            pl.BlockSpec((BM, F_CHUNK), lambda i, j: (j, 0)),
        ],
        out_specs=pl.BlockSpec((BM, BM), lambda i, j: (i, j)),
        out_shape=jax.ShapeDtypeStruct((M, N), jnp.float32),
    )(dy_c, w_c)


def _all_reduce(p):
    def body(p_ref, out_ref, rx, ry, rz, send_sems, recv_sems):
        x = lax.axis_index("x")
        y = lax.axis_index("y")
        z = lax.axis_index("z")
        px = (1 - x, y, z)
        py = (x, 1 - y, z)
        pz = (x, y, 1 - z)

        barrier_sem = pltpu.get_barrier_semaphore()
        for peer in (px, py, pz):
            pl.semaphore_signal(
                barrier_sem, inc=1,
                device_id=peer, device_id_type=pl.DeviceIdType.MESH,
            )
        pl.semaphore_wait(barrier_sem, 3)

        keep_x = x * H_X
        send_x = (1 - x) * H_X
        keep_y = keep_x + y * H_Y
        send_y = keep_x + (1 - y) * H_Y
        keep_z = keep_y + z * H_Z
        send_z = keep_y + (1 - z) * H_Z

        def exchange(src, dst, ph, peer):
            rdma = pltpu.make_async_remote_copy(
                src_ref=src, dst_ref=dst,
                send_sem=send_sems.at[ph], recv_sem=recv_sems.at[ph],
                device_id=peer, device_id_type=pl.DeviceIdType.MESH,
            )
            rdma.start()
            rdma.wait()

        exchange(p_ref.at[pl.ds(send_x, H_X)], rx, 0, px)
        out_ref[pl.ds(keep_x, H_X), :] = p_ref[pl.ds(keep_x, H_X), :] + rx[...]

        exchange(out_ref.at[pl.ds(send_y, H_Y)], ry, 1, py)
        out_ref[pl.ds(keep_y, H_Y), :] = out_ref[pl.ds(keep_y, H_Y), :] + ry[...]

        exchange(out_ref.at[pl.ds(send_z, H_Z)], rz, 2, pz)
        out_ref[pl.ds(keep_z, H_Z), :] = out_ref[pl.ds(keep_z, H_Z), :] + rz[...]

        exchange(out_ref.at[pl.ds(keep_z, H_Z)], out_ref.at[pl.ds(keep_z, H_Z)], 3, pz)
        exchange(out_ref.at[pl.ds(keep_y, H_Y)], out_ref.at[pl.ds(keep_y, H_Y)], 4, py)
        exchange(out_ref.at[pl.ds(keep_x, H_X)], out_ref.at[pl.ds(keep_x, H_X)], 5, px)

    return pl.pallas_call(
        body,
        out_shape=jax.ShapeDtypeStruct((M, N), jnp.float32),
        in_specs=[pl.BlockSpec(memory_space=pltpu.VMEM)],
        out_specs=pl.BlockSpec(memory_space=pltpu.VMEM),
        scratch_shapes=[
            pltpu.VMEM((H_X, N), jnp.float32),
            pltpu.VMEM((H_Y, N), jnp.float32),
            pltpu.VMEM((H_Z, N), jnp.float32),
            pltpu.SemaphoreType.DMA((6,)),
            pltpu.SemaphoreType.DMA((6,)),
        ],
        compiler_params=pltpu.CompilerParams(collective_id=0),
    )(p)


def kernel(dy, W):
    r = lax.axis_index("x") * 2 + lax.axis_index("z")
    dy_c = lax.dynamic_slice_in_dim(dy, r * F_CHUNK, F_CHUNK, axis=1)
    w_c = lax.dynamic_slice_in_dim(W, r * F_CHUNK, F_CHUNK, axis=1)
    return _all_reduce(_matmul(dy_c, w_c))


# device time: 211931 ns/iter; 1.9268x vs baseline; 1.9268x over previous
import jax
import jax.numpy as jnp
from jax import lax
from jax.experimental import pallas as pl
from jax.experimental.pallas import tpu as pltpu

M = 2048
N = 2048
F_CHUNK = 2048
BM = 512

H_X = M // 2
H_Y = M // 4
H_Z = M // 8


def _matmul(dy_c, w_c):
    def mm_body(dy_ref, w_ref, p_ref):
        p_ref[...] = lax.dot_general(
            dy_ref[...], w_ref[...],
            dimension_numbers=(((1,), (1,)), ((), ())),
            preferred_element_type=jnp.float32,
        )

    return pl.pallas_call(
        mm_body,
        grid=(M // BM, N // BM),
        in_specs=[
            pl.BlockSpec((BM, F_CHUNK), lambda i, j: (i, 0)),
            pl.BlockSpec((BM, F_CHUNK), lambda i, j: (j, 0)),
        ],
        out_specs=pl.BlockSpec((BM, BM), lambda i, j: (i, j)),
        out_shape=jax.ShapeDtypeStruct((M, N), jnp.float32),
    )(dy_c, w_c)


GROUPS = (
    (0, 768, ("x", "y", "z")),
    (768, 640, ("y", "z", "x")),
    (1408, 640, ("z", "x", "y")),
)
SCRATCH_ROWS = sum(s // 2 + s // 4 + s // 8 for _, s, _ in GROUPS)


def _all_reduce(p):
    def body(p_ref, out_ref, scratch, send_sems, recv_sems):
        x = lax.axis_index("x")
        y = lax.axis_index("y")
        z = lax.axis_index("z")
        coord = {"x": x, "y": y, "z": z}

        def peer_of(axis):
            return (
                1 - x if axis == "x" else x,
                1 - y if axis == "y" else y,
                1 - z if axis == "z" else z,
            )

        barrier_sem = pltpu.get_barrier_semaphore()
        for axis in ("x", "y", "z"):
            pl.semaphore_signal(
                barrier_sem, inc=1,
                device_id=peer_of(axis), device_id_type=pl.DeviceIdType.MESH,
            )
        pl.semaphore_wait(barrier_sem, 3)

        plans = []
        soff = 0
        for g0, rows, order in GROUPS:
            keep = g0
            phases = []
            for ph, axis in enumerate(order):
                h = rows >> (ph + 1)
                k = keep + coord[axis] * h
                snd = keep + (1 - coord[axis]) * h
                phases.append((axis, h, k, snd, soff))
                keep = k
                soff += h
            plans.append(phases)

        def start(src, dst, sem_idx, axis):
            rdma = pltpu.make_async_remote_copy(
                src_ref=src, dst_ref=dst,
                send_sem=send_sems.at[sem_idx], recv_sem=recv_sems.at[sem_idx],
                device_id=peer_of(axis), device_id_type=pl.DeviceIdType.MESH,
            )
            rdma.start()
            return rdma

        for ph in range(3):
            rdmas = []
            for g, phases in enumerate(plans):
                axis, h, k, snd, so = phases[ph]
                src = p_ref if ph == 0 else out_ref
                rdmas.append(
                    start(src.at[pl.ds(snd, h)], scratch.at[pl.ds(so, h)],
                          g * 3 + ph, axis)
                )
            for g, phases in enumerate(plans):
                axis, h, k, snd, so = phases[ph]
                src = p_ref if ph == 0 else out_ref
                rdmas[g].wait()
                out_ref[pl.ds(k, h), :] = (
                    src[pl.ds(k, h), :] + scratch[pl.ds(so, h), :]
                )

        for ph in range(2, -1, -1):
            rdmas = []
            for g, phases in enumerate(plans):
                axis, h, k, _snd, _so = phases[ph]
                rdmas.append(
                    start(out_ref.at[pl.ds(k, h)], out_ref.at[pl.ds(k, h)],
                          9 + g * 3 + ph, axis)
                )
            for r in rdmas:
                r.wait()

    return pl.pallas_call(
        body,
        out_shape=jax.ShapeDtypeStruct((M, N), jnp.float32),
        in_specs=[pl.BlockSpec(memory_space=pltpu.VMEM)],
        out_specs=pl.BlockSpec(memory_space=pltpu.VMEM),
        scratch_shapes=[
            pltpu.VMEM((SCRATCH_ROWS, N), jnp.float32),
            pltpu.SemaphoreType.DMA((18,)),
            pltpu.SemaphoreType.DMA((18,)),
        ],
        compiler_params=pltpu.CompilerParams(collective_id=0),
    )(p)


def kernel(dy, W):
    r = lax.axis_index("x") * 2 + lax.axis_index("z")
    dy_c = lax.dynamic_slice_in_dim(dy, r * F_CHUNK, F_CHUNK, axis=1)
    w_c = lax.dynamic_slice_in_dim(W, r * F_CHUNK, F_CHUNK, axis=1)
    return _all_reduce(_matmul(dy_c, w_c))


# device time: 190559 ns/iter; 2.1429x vs baseline; 1.1122x over previous
import jax
import jax.numpy as jnp
from jax import lax
from jax.experimental import pallas as pl
from jax.experimental.pallas import tpu as pltpu

M = 2048
N = 2048
F_CHUNK = 2048

GROUPS = (
    (0, 768, ("x", "y", "z")),
    (768, 640, ("y", "z", "x")),
    (1408, 640, ("z", "x", "y")),
)
SCRATCH_ROWS = sum(s // 2 + s // 4 + s // 8 for _, s, _ in GROUPS)


def kernel(dy, W):
    r = lax.axis_index("x") * 2 + lax.axis_index("z")
    dy_c = lax.dynamic_slice_in_dim(dy, r * F_CHUNK, F_CHUNK, axis=1)
    w_c = lax.dynamic_slice_in_dim(W, r * F_CHUNK, F_CHUNK, axis=1)

    def body(dy_ref, w_ref, out_ref, scratch, send_sems, recv_sems):
        x = lax.axis_index("x")
        y = lax.axis_index("y")
        z = lax.axis_index("z")
        coord = {"x": x, "y": y, "z": z}

        def peer_of(axis):
            return (
                1 - x if axis == "x" else x,
                1 - y if axis == "y" else y,
                1 - z if axis == "z" else z,
            )

        barrier_sem = pltpu.get_barrier_semaphore()
        for axis in ("x", "y", "z"):
            pl.semaphore_signal(
                barrier_sem, inc=1,
                device_id=peer_of(axis), device_id_type=pl.DeviceIdType.MESH,
            )
        pl.semaphore_wait(barrier_sem, 3)

        plans = []
        soff = 0
        for g0, rows, order in GROUPS:
            keep = g0
            phases = []
            for ph, axis in enumerate(order):
                h = rows >> (ph + 1)
                k = keep + coord[axis] * h
                snd = keep + (1 - coord[axis]) * h
                phases.append((axis, h, k, snd, soff))
                keep = k
                soff += h
            plans.append(phases)

        def start(src, dst, sem_idx, axis):
            rdma = pltpu.make_async_remote_copy(
                src_ref=src, dst_ref=dst,
                send_sem=send_sems.at[sem_idx], recv_sem=recv_sems.at[sem_idx],
                device_id=peer_of(axis), device_id_type=pl.DeviceIdType.MESH,
            )
            rdma.start()
            return rdma

        def gemm_rows(off, h):
            out_ref[pl.ds(off, h), :] = lax.dot_general(
                dy_ref[pl.ds(off, h), :], w_ref[...],
                dimension_numbers=(((1,), (1,)), ((), ())),
                preferred_element_type=jnp.float32,
            )

        rdmas = []
        for g, phases in enumerate(plans):
            axis, h, _k, snd, so = phases[0]
            gemm_rows(snd, h)
            rdmas.append(
                start(out_ref.at[pl.ds(snd, h)], scratch.at[pl.ds(so, h)],
                      g * 3, axis)
            )
        for g, phases in enumerate(plans):
            _axis, h, k, _snd, _so = phases[0]
            gemm_rows(k, h)

        for ph in range(3):
            next_rdmas = []
            for g, phases in enumerate(plans):
                axis, h, k, snd, so = phases[ph]
                rdmas[g].wait()
                out_ref[pl.ds(k, h), :] = (
                    out_ref[pl.ds(k, h), :] + scratch[pl.ds(so, h), :]
                )
                if ph < 2:
                    naxis, nh, _nk, nsnd, nso = phases[ph + 1]
                    next_rdmas.append(
                        start(out_ref.at[pl.ds(nsnd, nh)],
                              scratch.at[pl.ds(nso, nh)],
                              g * 3 + ph + 1, naxis)
                    )
            rdmas = next_rdmas

        for ph in range(2, -1, -1):
            rdmas = []
            for g, phases in enumerate(plans):
                axis, h, k, _snd, _so = phases[ph]
                rdmas.append(
                    start(out_ref.at[pl.ds(k, h)], out_ref.at[pl.ds(k, h)],
                          9 + g * 3 + ph, axis)
                )
            for rdma in rdmas:
                rdma.wait()

    return pl.pallas_call(
        body,
        out_shape=jax.ShapeDtypeStruct((M, N), jnp.float32),
        in_specs=[
            pl.BlockSpec(memory_space=pltpu.VMEM),
            pl.BlockSpec(memory_space=pltpu.VMEM),
        ],
        out_specs=pl.BlockSpec(memory_space=pltpu.VMEM),
        scratch_shapes=[
            pltpu.VMEM((SCRATCH_ROWS, N), jnp.float32),
            pltpu.SemaphoreType.DMA((18,)),
            pltpu.SemaphoreType.DMA((18,)),
        ],
        compiler_params=pltpu.CompilerParams(
            collective_id=0,
            vmem_limit_bytes=63 * 1024 * 1024,
        ),
    )(dy_c, w_c)
